# baseline (speedup 1.0000x reference)
import functools

import jax
import jax.numpy as jnp
import numpy as _np
from jax import lax
from jax.experimental import pallas as pl
from jax.experimental.pallas import tpu as pltpu
from jax.experimental.pallas import tpu_sc as plsc

N_NODES = 10000
N_EDGES = 320000
D = 128

NC = 2
NS = 16
NW = NC * NS

CHUNK = 128
N_CHUNKS = N_EDGES // CHUNK
NSLICE = 4
SCH = N_CHUNKS // NSLICE
SE = SCH * CHUNK
BASE_CH = SCH // NW
EXTRA = SCH % NW
IDXR = 24
NPAD = 10240
STRIPE = NPAD // NS

KB = 3
BROWS = KB * CHUNK
NFULL = BASE_CH // KB

_mesh = lambda: plsc.VectorSubcoreMesh(core_axis_name="c", subcore_axis_name="s")


def _worker_range(wid):
    nch = BASE_CH + jnp.where(wid < EXTRA, 1, 0)
    start = wid * BASE_CH + jnp.minimum(wid, EXTRA)
    return start, nch


def _gelu(x):
    return 0.5 * x * (1.0 + lax.erf(x * 0.7071067811865476))


def _xw_body(x_ref, w_ref, o_ref):
    o_ref[...] = jnp.dot(x_ref[...], w_ref[...], preferred_element_type=jnp.float32)


def _project_nodes(x_feat, w1t):
    return pl.pallas_call(
        _xw_body,
        out_shape=jax.ShapeDtypeStruct((N_NODES, D), jnp.float32),
    )(x_feat, w1t)


EB = 4000


def _combine_body(e_ref, g_ref, bs_ref, w_ref, b1_ref, o_ref):
    pre = jnp.dot(e_ref[...], w_ref[...], preferred_element_type=jnp.float32)
    pre = pre + g_ref[...] + b1_ref[...]
    o_ref[...] = _gelu(pre) * bs_ref[...]


def _combine(s, edge_attr, g_s, bases, w1t, b1_row):
    loc = lambda i: (i, 0)
    glo = lambda i: (s * (SE // EB) + i, 0)
    fix = lambda i: (0, 0)
    return pl.pallas_call(
        _combine_body,
        grid=(SE // EB,),
        in_specs=[
            pl.BlockSpec((EB, D), glo),
            pl.BlockSpec((EB, D), loc),
            pl.BlockSpec((EB, D), glo),
            pl.BlockSpec((D, D), fix),
            pl.BlockSpec((1, D), fix),
        ],
        out_specs=pl.BlockSpec((EB, D), loc),
        out_shape=jax.ShapeDtypeStruct((SE, D), jnp.float32),
    )(edge_attr, g_s, bases, w1t, b1_row)


RB = 1280


def _reduce_body(a0, a1, a2, a3, o_ref):
    o_ref[...] = (a0[0] + a0[1] + a1[0] + a1[1]) + (a2[0] + a2[1] + a3[0] + a3[1])


def _reduce_accs(accs):
    blk = pl.BlockSpec((NC, RB, D), lambda i: (0, i, 0))
    return pl.pallas_call(
        _reduce_body,
        grid=(NPAD // RB,),
        in_specs=[blk] * NSLICE,
        out_specs=pl.BlockSpec((RB, D), lambda i: (i, 0)),
        out_shape=jax.ShapeDtypeStruct((NPAD, D), jnp.float32),
    )(*accs)


def _ffn_body(x_ref, a_ref, w2_ref, b2_ref, g1_ref, be1_ref,
              w3_ref, b3_ref, g2_ref, be2_ref, o_ref):
    x = x_ref[...] + a_ref[:N_NODES]
    h = jnp.dot(x, w2_ref[...], preferred_element_type=jnp.float32) + b2_ref[...]
    mean = jnp.mean(h, axis=0, keepdims=True)
    var = jnp.mean((h - mean) ** 2, axis=0, keepdims=True)
    h = (h - mean) / jnp.sqrt(var + 1e-5) * g1_ref[...] + be1_ref[...]
    h = _gelu(h)
    h = jnp.dot(h, w3_ref[...], preferred_element_type=jnp.float32) + b3_ref[...]
    mean = jnp.mean(h, axis=0, keepdims=True)
    var = jnp.mean((h - mean) ** 2, axis=0, keepdims=True)
    h = (h - mean) / jnp.sqrt(var + 1e-5) * g2_ref[...] + be2_ref[...]
    h = _gelu(h)
    o_ref[...] = x + h


def _ffn(x_feat, aggr, w2t, b2r, g1r, be1r, w3t, b3r, g2r, be2r):
    return pl.pallas_call(
        _ffn_body,
        out_shape=jax.ShapeDtypeStruct((N_NODES, D), jnp.float32),
    )(x_feat, aggr, w2t, b2r, g1r, be1r, w3t, b3r, g2r, be2r)


def _make_gather(sw_off):
    def body(table_hbm, idx_hbm, out_hbm, idx_v, rows_a, rows_b, gsa, gsb):
        cid = lax.axis_index("c")
        sid = lax.axis_index("s")
        wid = sid * NC + cid
        start, nch = _worker_range(wid)
        pltpu.sync_copy(idx_hbm.at[sw_off + wid], idx_v)

        def issue(b, rows, sem):
            for k in range(KB):
                pltpu.async_copy(table_hbm.at[idx_v.at[b * KB + k]],
                                 rows.at[pl.ds(k * CHUNK, CHUNK)], sem)

        def wait(rows, sem):
            pltpu.make_async_copy(out_hbm.at[pl.ds(0, BROWS)], rows, sem).wait()

        def store(b, rows):
            ofs = pl.multiple_of((start + b * KB) * CHUNK, CHUNK)
            pltpu.sync_copy(rows, out_hbm.at[pl.ds(ofs, BROWS)])

        issue(0, rows_a, gsa)

        def loop(b, carry):
            even = b % 2 == 0

            @pl.when(even)
            def _():
                wait(rows_a, gsa)

                @pl.when(b + 1 < NFULL)
                def _():
                    issue(b + 1, rows_b, gsb)
                store(b, rows_a)

            @pl.when(jnp.logical_not(even))
            def _():
                wait(rows_b, gsb)

                @pl.when(b + 1 < NFULL)
                def _():
                    issue(b + 1, rows_a, gsa)
                store(b, rows_b)

            return carry

        lax.fori_loop(0, NFULL, loop, 0)

        def tail(j, carry):
            ofs = pl.multiple_of((start + j) * CHUNK, CHUNK)
            pltpu.async_copy(table_hbm.at[idx_v.at[j]],
                             rows_a.at[pl.ds(0, CHUNK)], gsa).wait()
            pltpu.sync_copy(rows_a.at[pl.ds(0, CHUNK)], out_hbm.at[pl.ds(ofs, CHUNK)])
            return carry

        lax.fori_loop(NFULL * KB, nch, tail, 0)

    return body


def _gather(s, table, idx_pad):
    k = functools.partial(
        pl.kernel,
        out_type=jax.ShapeDtypeStruct((SE, D), jnp.float32),
        mesh=_mesh(),
        scratch_types=[
            pltpu.VMEM((IDXR, CHUNK), jnp.int32),
            pltpu.VMEM((BROWS, D), jnp.float32),
            pltpu.VMEM((BROWS, D), jnp.float32),
            pltpu.SemaphoreType.DMA,
            pltpu.SemaphoreType.DMA,
        ],
        name=f"edge_gather_s{s}",
    )(_make_gather(s * NW))
    return k(table, idx_pad)


def _make_scatter(sw_off):
    def body(v_hbm, dst_hbm, zeros_hbm, out_hbm, idx_v, rows_a, rows_b, acc_sh, lsa, lsb):
        cid = lax.axis_index("c")
        sid = lax.axis_index("s")
        wid = sid * NC + cid
        pltpu.sync_copy(zeros_hbm.at[pl.ds(sid * STRIPE, STRIPE)],
                        acc_sh.at[pl.ds(sid * STRIPE, STRIPE)])
        plsc.subcore_barrier()

        start, nch = _worker_range(wid)
        pltpu.sync_copy(dst_hbm.at[sw_off + wid], idx_v)

        def issue(j, rows, sem):
            ofs = pl.multiple_of((start + j) * CHUNK, CHUNK)
            pltpu.async_copy(v_hbm.at[pl.ds(ofs, CHUNK)], rows, sem)

        def wait(rows, sem):
            pltpu.make_async_copy(v_hbm.at[pl.ds(0, CHUNK)], rows, sem).wait()

        def scat(j, rows):
            pltpu.sync_copy(rows, acc_sh.at[idx_v.at[j]], add=True)

        issue(0, rows_a, lsa)

        def loop(j, carry):
            even = j % 2 == 0

            @pl.when(even)
            def _():
                wait(rows_a, lsa)

                @pl.when(j + 1 < nch)
                def _():
                    issue(j + 1, rows_b, lsb)
                scat(j, rows_a)

            @pl.when(jnp.logical_not(even))
            def _():
                wait(rows_b, lsb)

                @pl.when(j + 1 < nch)
                def _():
                    issue(j + 1, rows_a, lsa)
                scat(j, rows_b)

            return carry

        lax.fori_loop(0, nch, loop, 0)
        plsc.subcore_barrier()
        pltpu.sync_copy(acc_sh.at[pl.ds(sid * STRIPE, STRIPE)],
                        out_hbm.at[cid, pl.ds(sid * STRIPE, STRIPE)])

    return body


def _scatter(s, v_s, dst_pad, zeros):
    k = functools.partial(
        pl.kernel,
        out_type=jax.ShapeDtypeStruct((NC, NPAD, D), jnp.float32),
        mesh=_mesh(),
        scratch_types=[
            pltpu.VMEM((IDXR, CHUNK), jnp.int32),
            pltpu.VMEM((CHUNK, D), jnp.float32),
            pltpu.VMEM((CHUNK, D), jnp.float32),
            pltpu.VMEM_SHARED((NPAD, D), jnp.float32),
            pltpu.SemaphoreType.DMA,
            pltpu.SemaphoreType.DMA,
        ],
        name=f"edge_scatter_s{s}",
    )(_make_scatter(s * NW))
    return k(v_s, dst_pad, zeros)


def _w_rows():
    rows = _np.zeros((NSLICE * NW, IDXR), _np.int64)
    for s in range(NSLICE):
        for w in range(NW):
            start = w * BASE_CH + min(w, EXTRA)
            r = _np.minimum(start + _np.arange(IDXR), SCH - 1)
            rows[s * NW + w] = s * SCH + r
    return rows


_W_ROWS = _w_rows()


def _pad_idx(idx):
    idx2 = idx.reshape(N_CHUNKS, CHUNK)
    return jnp.take(idx2, jnp.asarray(_W_ROWS), axis=0)


def kernel(x_feat, edge_attr, bases, edge_index, W1, b1, W2, b2, g1, be1, W3, b3, g2, be2):
    src_pad = _pad_idx(edge_index[0])
    dst_pad = _pad_idx(edge_index[1])
    zeros = jnp.zeros((NPAD, D), jnp.float32)
    w1t = W1.T

    xw = _project_nodes(x_feat, w1t)
    accs = []
    for s in range(NSLICE):
        g_s = _gather(s, xw, src_pad)
        v_s = _combine(s, edge_attr, g_s, bases, w1t, b1.reshape(1, D))
        accs.append(_scatter(s, v_s, dst_pad, zeros))
    aggr = _reduce_accs(accs)
    out = _ffn(x_feat, aggr, W2.T, b2.reshape(1, D), g1.reshape(1, D),
               be1.reshape(1, D), W3.T, b3.reshape(1, D), g2.reshape(1, D),
               be2.reshape(1, D))
    return out


# speedup vs baseline: 1.1346x; 1.1346x over previous
import functools

import jax
import jax.numpy as jnp
import numpy as _np
from jax import lax
from jax.experimental import pallas as pl
from jax.experimental.pallas import tpu as pltpu
from jax.experimental.pallas import tpu_sc as plsc

N_NODES = 10000
N_EDGES = 320000
D = 128

NC = 2
NS = 16
NW = NC * NS

CHUNK = 128
N_CHUNKS = N_EDGES // CHUNK
NSLICE = 2
SCH = N_CHUNKS // NSLICE
SE = SCH * CHUNK
BASE_CH = SCH // NW
EXTRA = SCH % NW
IDXR = 40
NPAD = 10240
STRIPE = NPAD // NS

KB = 3
BROWS = KB * CHUNK
NFULL = BASE_CH // KB

_mesh = lambda: plsc.VectorSubcoreMesh(core_axis_name="c", subcore_axis_name="s")


def _worker_range(wid):
    nch = BASE_CH + jnp.where(wid < EXTRA, 1, 0)
    start = wid * BASE_CH + jnp.minimum(wid, EXTRA)
    return start, nch


def _gelu(x):
    return 0.5 * x * (1.0 + lax.erf(x * 0.7071067811865476))


def _xw_body(x_ref, w_ref, o_ref):
    o_ref[...] = jnp.dot(x_ref[...], w_ref[...], preferred_element_type=jnp.float32)


def _project_nodes(x_feat, w1t):
    return pl.pallas_call(
        _xw_body,
        out_shape=jax.ShapeDtypeStruct((N_NODES, D), jnp.float32),
    )(x_feat, w1t)


EB = 4000


def _combine_body(e_ref, g_ref, bs_ref, w_ref, b1_ref, o_ref):
    pre = jnp.dot(e_ref[...], w_ref[...], preferred_element_type=jnp.float32)
    pre = pre + g_ref[...] + b1_ref[...]
    o_ref[...] = _gelu(pre) * bs_ref[...]


def _combine(s, edge_attr, g_s, bases, w1t, b1_row):
    loc = lambda i: (i, 0)
    glo = lambda i: (s * (SE // EB) + i, 0)
    fix = lambda i: (0, 0)
    return pl.pallas_call(
        _combine_body,
        grid=(SE // EB,),
        in_specs=[
            pl.BlockSpec((EB, D), glo),
            pl.BlockSpec((EB, D), loc),
            pl.BlockSpec((EB, D), glo),
            pl.BlockSpec((D, D), fix),
            pl.BlockSpec((1, D), fix),
        ],
        out_specs=pl.BlockSpec((EB, D), loc),
        out_shape=jax.ShapeDtypeStruct((SE, D), jnp.float32),
    )(edge_attr, g_s, bases, w1t, b1_row)


def _ffn_body(x_ref, a0_ref, a1_ref, w2_ref, b2_ref, g1_ref, be1_ref,
              w3_ref, b3_ref, g2_ref, be2_ref, o_ref):
    aggr = (a0_ref[0, :N_NODES] + a0_ref[1, :N_NODES]) + \
           (a1_ref[0, :N_NODES] + a1_ref[1, :N_NODES])
    x = x_ref[...] + aggr
    h = jnp.dot(x, w2_ref[...], preferred_element_type=jnp.float32) + b2_ref[...]
    mean = jnp.mean(h, axis=0, keepdims=True)
    var = jnp.mean((h - mean) ** 2, axis=0, keepdims=True)
    h = (h - mean) / jnp.sqrt(var + 1e-5) * g1_ref[...] + be1_ref[...]
    h = _gelu(h)
    h = jnp.dot(h, w3_ref[...], preferred_element_type=jnp.float32) + b3_ref[...]
    mean = jnp.mean(h, axis=0, keepdims=True)
    var = jnp.mean((h - mean) ** 2, axis=0, keepdims=True)
    h = (h - mean) / jnp.sqrt(var + 1e-5) * g2_ref[...] + be2_ref[...]
    h = _gelu(h)
    o_ref[...] = x + h


def _ffn(x_feat, accs, w2t, b2r, g1r, be1r, w3t, b3r, g2r, be2r):
    return pl.pallas_call(
        _ffn_body,
        out_shape=jax.ShapeDtypeStruct((N_NODES, D), jnp.float32),
    )(x_feat, *accs, w2t, b2r, g1r, be1r, w3t, b3r, g2r, be2r)


def _make_gather(sw_off):
    def body(table_hbm, idx_hbm, out_hbm, idx_v, rows_a, rows_b, gsa, gsb):
        cid = lax.axis_index("c")
        sid = lax.axis_index("s")
        wid = sid * NC + cid
        start, nch = _worker_range(wid)
        pltpu.sync_copy(idx_hbm.at[sw_off + wid], idx_v)

        def issue(b, rows, sem):
            for k in range(KB):
                pltpu.async_copy(table_hbm.at[idx_v.at[b * KB + k]],
                                 rows.at[pl.ds(k * CHUNK, CHUNK)], sem)

        def wait(rows, sem):
            pltpu.make_async_copy(out_hbm.at[pl.ds(0, BROWS)], rows, sem).wait()

        def store(b, rows):
            ofs = pl.multiple_of((start + b * KB) * CHUNK, CHUNK)
            pltpu.sync_copy(rows, out_hbm.at[pl.ds(ofs, BROWS)])

        issue(0, rows_a, gsa)

        def loop(b, carry):
            even = b % 2 == 0

            @pl.when(even)
            def _():
                wait(rows_a, gsa)

                @pl.when(b + 1 < NFULL)
                def _():
                    issue(b + 1, rows_b, gsb)
                store(b, rows_a)

            @pl.when(jnp.logical_not(even))
            def _():
                wait(rows_b, gsb)

                @pl.when(b + 1 < NFULL)
                def _():
                    issue(b + 1, rows_a, gsa)
                store(b, rows_b)

            return carry

        lax.fori_loop(0, NFULL, loop, 0)

        def tail(j, carry):
            ofs = pl.multiple_of((start + j) * CHUNK, CHUNK)
            pltpu.async_copy(table_hbm.at[idx_v.at[j]],
                             rows_a.at[pl.ds(0, CHUNK)], gsa).wait()
            pltpu.sync_copy(rows_a.at[pl.ds(0, CHUNK)], out_hbm.at[pl.ds(ofs, CHUNK)])
            return carry

        lax.fori_loop(NFULL * KB, nch, tail, 0)

    return body


def _gather(s, table, idx_pad):
    k = functools.partial(
        pl.kernel,
        out_type=jax.ShapeDtypeStruct((SE, D), jnp.float32),
        mesh=_mesh(),
        scratch_types=[
            pltpu.VMEM((IDXR, CHUNK), jnp.int32),
            pltpu.VMEM((BROWS, D), jnp.float32),
            pltpu.VMEM((BROWS, D), jnp.float32),
            pltpu.SemaphoreType.DMA,
            pltpu.SemaphoreType.DMA,
        ],
        name=f"edge_gather_s{s}",
    )(_make_gather(s * NW))
    return k(table, idx_pad)


def _make_scatter(sw_off):
    def body(v_hbm, dst_hbm, out_hbm, idx_v, rows_a, rows_b, acc_sh, lsa, lsb):
        cid = lax.axis_index("c")
        sid = lax.axis_index("s")
        wid = sid * NC + cid

        zero16 = jnp.zeros((16,), jnp.float32)

        def zrow(r, carry):
            for k in range(D // 16):
                rows_a[r, pl.ds(k * 16, 16)] = zero16
            return carry

        lax.fori_loop(0, CHUNK, zrow, 0)
        for t in range(STRIPE // CHUNK):
            ofs = pl.multiple_of(sid * STRIPE + t * CHUNK, CHUNK)
            pltpu.sync_copy(rows_a, acc_sh.at[pl.ds(ofs, CHUNK)])
        plsc.subcore_barrier()

        start, nch = _worker_range(wid)
        pltpu.sync_copy(dst_hbm.at[sw_off + wid], idx_v)

        def issue(j, rows, sem):
            ofs = pl.multiple_of((start + j) * CHUNK, CHUNK)
            pltpu.async_copy(v_hbm.at[pl.ds(ofs, CHUNK)], rows, sem)

        def wait(rows, sem):
            pltpu.make_async_copy(v_hbm.at[pl.ds(0, CHUNK)], rows, sem).wait()

        def scat(j, rows):
            pltpu.sync_copy(rows, acc_sh.at[idx_v.at[j]], add=True)

        issue(0, rows_a, lsa)

        def loop(j, carry):
            even = j % 2 == 0

            @pl.when(even)
            def _():
                wait(rows_a, lsa)

                @pl.when(j + 1 < nch)
                def _():
                    issue(j + 1, rows_b, lsb)
                scat(j, rows_a)

            @pl.when(jnp.logical_not(even))
            def _():
                wait(rows_b, lsb)

                @pl.when(j + 1 < nch)
                def _():
                    issue(j + 1, rows_a, lsa)
                scat(j, rows_b)

            return carry

        lax.fori_loop(0, nch, loop, 0)
        plsc.subcore_barrier()
        pltpu.sync_copy(acc_sh.at[pl.ds(sid * STRIPE, STRIPE)],
                        out_hbm.at[cid, pl.ds(sid * STRIPE, STRIPE)])

    return body


def _scatter(s, v_s, dst_pad):
    k = functools.partial(
        pl.kernel,
        out_type=jax.ShapeDtypeStruct((NC, NPAD, D), jnp.float32),
        mesh=_mesh(),
        scratch_types=[
            pltpu.VMEM((IDXR, CHUNK), jnp.int32),
            pltpu.VMEM((CHUNK, D), jnp.float32),
            pltpu.VMEM((CHUNK, D), jnp.float32),
            pltpu.VMEM_SHARED((NPAD, D), jnp.float32),
            pltpu.SemaphoreType.DMA,
            pltpu.SemaphoreType.DMA,
        ],
        name=f"edge_scatter_s{s}",
    )(_make_scatter((NSLICE + s) * NW))
    return k(v_s, dst_pad)


def _w_rows():
    rows = _np.zeros((NSLICE * NW, IDXR), _np.int64)
    for s in range(NSLICE):
        for w in range(NW):
            start = w * BASE_CH + min(w, EXTRA)
            r = _np.minimum(start + _np.arange(IDXR), SCH - 1)
            rows[s * NW + w] = s * SCH + r
    return _np.concatenate([rows, rows + N_CHUNKS], axis=0)


_W_ROWS = _w_rows()


def kernel(x_feat, edge_attr, bases, edge_index, W1, b1, W2, b2, g1, be1, W3, b3, g2, be2):
    idx2 = edge_index.reshape(2 * N_CHUNKS, CHUNK)
    idx_pad = jnp.take(idx2, jnp.asarray(_W_ROWS), axis=0)
    w1t = W1.T

    xw = _project_nodes(x_feat, w1t)
    accs = []
    for s in range(NSLICE):
        g_s = _gather(s, xw, idx_pad)
        v_s = _combine(s, edge_attr, g_s, bases, w1t, b1.reshape(1, D))
        accs.append(_scatter(s, v_s, idx_pad))
    out = _ffn(x_feat, accs, W2.T, b2.reshape(1, D), g1.reshape(1, D),
               be1.reshape(1, D), W3.T, b3.reshape(1, D), g2.reshape(1, D),
               be2.reshape(1, D))
    return out
